# baseline (speedup 1.0000x reference)
import functools

import jax
import jax.numpy as jnp
from jax import lax
from jax.experimental import pallas as pl
from jax.experimental.pallas import tpu as pltpu
from jax.experimental.pallas import tpu_sc as plsc

N = 8192
K = 16
HID = 64
FEAT = 64
TW = 128


BM = 128


def _knn_body(p1_ref, p0t_ref, idx_ref, d2_ref):
    p1 = p1_ref[...]
    p0t = p0t_ref[...]
    dot = jnp.dot(p1, p0t, preferred_element_type=jnp.float32)
    n1 = jnp.sum(p1 * p1, axis=1, keepdims=True)
    n0 = jnp.sum(p0t * p0t, axis=0, keepdims=True)
    d2_ref[...] = n1 + n0 - 2.0 * dot
    iota = lax.broadcasted_iota(jnp.int32, (BM, N), 1)
    inf = jnp.float32(jnp.inf)
    cols = []
    t = jnp.full((BM, 1), -inf, jnp.float32)
    it = jnp.full((BM, 1), -1, jnp.int32)
    for _ in range(K):
        vals = d2_ref[...]
        active = (vals > t) | ((vals == t) & (iota > it))
        m = jnp.min(jnp.where(active, vals, inf), axis=1, keepdims=True)
        ij = jnp.min(jnp.where(active & (vals == m), iota, N),
                     axis=1, keepdims=True)
        cols.append(ij)
        t, it = m, ij
    idx_ref[...] = jnp.concatenate(cols, axis=1)


def _knn(p1p, p0tp):
    return pl.pallas_call(
        _knn_body,
        grid=(N // BM,),
        in_specs=[
            pl.BlockSpec((BM, 8), lambda i: (i, 0)),
            pl.BlockSpec((8, N), lambda i: (0, 0)),
        ],
        out_specs=pl.BlockSpec((BM, K), lambda i: (i, 0)),
        out_shape=jax.ShapeDtypeStruct((N, K), jnp.int32),
        scratch_shapes=[pltpu.VMEM((BM, N), jnp.float32)],
    )(p1p, p0tp)


TOT = N * K
CH = 512


def _gather_sc(table, idx_flat):
    info = plsc.get_sparse_core_info()
    nw = info.num_cores * info.num_subcores
    bpw = TOT // nw
    nch = bpw // CH
    mesh = plsc.VectorSubcoreMesh(core_axis_name="c", subcore_axis_name="s")

    @functools.partial(
        pl.kernel,
        mesh=mesh,
        out_type=jax.ShapeDtypeStruct((TOT, TW), jnp.float32),
        scratch_types=[
            pltpu.VMEM((CH,), jnp.int32),
            pltpu.VMEM((CH, TW), jnp.float32),
            pltpu.SemaphoreType.DMA,
        ],
    )
    def k(table_hbm, idx_hbm, out_hbm, idx_v, rows_v, sem):
        wid = lax.axis_index("s") * info.num_cores + lax.axis_index("c")
        base0 = wid * bpw

        def body(i, carry):
            base = base0 + i * CH
            pltpu.sync_copy(idx_hbm.at[pl.ds(base, CH)], idx_v)
            pltpu.async_copy(table_hbm.at[idx_v], rows_v, sem).wait()
            pltpu.sync_copy(rows_v, out_hbm.at[pl.ds(base, CH)])
            return carry

        lax.fori_loop(0, nch, body, 0)

    return k(table, idx_flat)


BM3 = 512


def _mlp_body(g_ref, p1_ref, feat_ref, wbig_ref, w3p_ref, wf_ref,
              wh1f_ref, wh1h_ref, c_ref, out_ref):
    gb = g_ref[...]
    y = jnp.dot(gb, wbig_ref[...], preferred_element_type=jnp.float32)
    p1b = p1_ref[...]
    pcon = jnp.dot(p1b, w3p_ref[...], preferred_element_type=jnp.float32)
    coords = gb[:, 64:67].reshape(BM3, K, 3)
    rela = coords - p1b[:, :3][:, None, :]
    dist = jnp.sqrt(jnp.sum(rela * rela, axis=-1, keepdims=True))
    wdist = c_ref[1:2, :].reshape(1, 1, 192)
    y3 = y.reshape(BM3, K, 192) - pcon[:, None, :] + dist * wdist
    ymax = jnp.max(y3, axis=1) + c_ref[0:1, :]
    featb = feat_ref[...]
    frz = jnp.dot(featb, wf_ref[...], preferred_element_type=jnp.float32)
    gate_r = jax.nn.sigmoid(ymax[:, 0:64] + frz[:, 0:64])
    gate_z = jax.nn.sigmoid(ymax[:, 64:128] + frz[:, 64:128])
    h10 = ymax[:, 128:192]
    h11 = jnp.tanh(
        jnp.dot(featb, wh1f_ref[...], preferred_element_type=jnp.float32)
        + jnp.dot(gate_r * h10, wh1h_ref[...], preferred_element_type=jnp.float32)
        + c_ref[2:3, 0:64])
    out_ref[...] = gate_z * h10 + (1.0 - gate_z) * h11


def _mlp(g, p1p, feat_t, wbig, w3p, wf, wh1f, wh1h, consts):
    return pl.pallas_call(
        _mlp_body,
        grid=(N // BM3,),
        in_specs=[
            pl.BlockSpec((BM3 * K, TW), lambda i: (i, 0)),
            pl.BlockSpec((BM3, 8), lambda i: (i, 0)),
            pl.BlockSpec((BM3, FEAT), lambda i: (i, 0)),
            pl.BlockSpec((TW, 192), lambda i: (0, 0)),
            pl.BlockSpec((8, 192), lambda i: (0, 0)),
            pl.BlockSpec((FEAT, 128), lambda i: (0, 0)),
            pl.BlockSpec((64, 64), lambda i: (0, 0)),
            pl.BlockSpec((64, 64), lambda i: (0, 0)),
            pl.BlockSpec((8, 192), lambda i: (0, 0)),
        ],
        out_specs=pl.BlockSpec((BM3, HID), lambda i: (i, 0)),
        out_shape=jax.ShapeDtypeStruct((N, HID), jnp.float32),
    )(g, p1p, feat_t, wbig, w3p, wf, wh1f, wh1h, consts)


def _pack_weights(W_R, b_R, W_Z, b_Z, W_H0, b_H0, W_H1, b_H1):
    wbig = jnp.zeros((TW, 192), jnp.float32)
    wbig = wbig.at[0:64, 0:64].set(W_R[4:68])
    wbig = wbig.at[64:67, 0:64].set(W_R[0:3])
    wbig = wbig.at[0:64, 64:128].set(W_Z[4:68])
    wbig = wbig.at[64:67, 64:128].set(W_Z[0:3])
    wbig = wbig.at[0:64, 128:192].set(W_H0[4:68])
    wbig = wbig.at[64:67, 128:192].set(W_H0[0:3])
    w3p = jnp.zeros((8, 192), jnp.float32)
    w3p = w3p.at[0:3, 0:64].set(W_R[0:3])
    w3p = w3p.at[0:3, 64:128].set(W_Z[0:3])
    w3p = w3p.at[0:3, 128:192].set(W_H0[0:3])
    wdist = jnp.concatenate([W_R[3], W_Z[3], W_H0[3]], axis=0)
    bcat = jnp.concatenate([b_R, b_Z, b_H0], axis=0)
    consts = jnp.zeros((8, 192), jnp.float32)
    consts = consts.at[0, :].set(bcat)
    consts = consts.at[1, :].set(wdist)
    consts = consts.at[2, 0:64].set(b_H1)
    wf = jnp.concatenate([W_R[68:132], W_Z[68:132]], axis=1)
    wh1f = W_H1[0:64]
    wh1h = W_H1[64:128]
    return wbig, w3p, wf, wh1f, wh1h, consts


def kernel(H0, points0, points1, contents1, motions1,
           W_R, b_R, W_Z, b_Z, W_H0, b_H0, W_H1, b_H1):
    p0t = points0[0]
    p1 = points1[0].T
    p1p = jnp.concatenate([p1, jnp.zeros((N, 5), jnp.float32)], axis=1)
    p0tp = jnp.concatenate([p0t, jnp.zeros((5, N), jnp.float32)], axis=0)

    nn_idx = _knn(p1p, p0tp)

    table = jnp.concatenate(
        [H0[0].T, p0t.T, jnp.zeros((N, TW - HID - 3), jnp.float32)], axis=1)
    g = _gather_sc(table, nn_idx.reshape(-1))

    feat_t = jnp.concatenate([contents1[0], motions1[0]], axis=0).T
    wbig, w3p, wf, wh1f, wh1h, consts = _pack_weights(
        W_R, b_R, W_Z, b_Z, W_H0, b_H0, W_H1, b_H1)
    out = _mlp(g, p1p, feat_t, wbig, w3p, wf, wh1f, wh1h, consts)
    return out.T[None]


# speedup vs baseline: 1.4377x; 1.4377x over previous
import functools

import jax
import jax.numpy as jnp
from jax import lax
from jax.experimental import pallas as pl
from jax.experimental.pallas import tpu as pltpu
from jax.experimental.pallas import tpu_sc as plsc

N = 8192
K = 16
HID = 64
FEAT = 64
TW = 128


BM = 128


NS = 64
NL = 128
RP = 4


def _knn_body(p1_ref, p0t_ref, idx_ref, d2_ref):
    p1 = p1_ref[...]
    p0t = p0t_ref[...]
    dot = jnp.dot(p1, p0t, preferred_element_type=jnp.float32)
    n1 = jnp.sum(p1 * p1, axis=1, keepdims=True)
    n0 = jnp.sum(p0t * p0t, axis=0, keepdims=True)
    d2_ref[...] = (n1 + n0 - 2.0 * dot).reshape(BM, NS, NL)
    inf = jnp.float32(jnp.inf)
    iota_s = lax.broadcasted_iota(jnp.int32, (BM, NS, NL), 1)
    iota_l = lax.broadcasted_iota(jnp.int32, (BM, NS, NL), 2)
    idx3 = iota_s * NL + iota_l
    pvs, pis, pss = [], [], []
    for _ in range(RP):
        v = d2_ref[...]
        ml = jnp.min(v, axis=1, keepdims=True)
        ss = jnp.min(jnp.where(v == ml, iota_s, NS), axis=1, keepdims=True)
        pvs.append(ml)
        pss.append(ss)
        pis.append(ss * NL + iota_l[:, :1, :])
        d2_ref[...] = jnp.where(iota_s == ss, inf, v)
    pv = jnp.concatenate(pvs, axis=1)
    pi = jnp.concatenate(pis, axis=1)
    t = jnp.full((BM, 1, 1), -inf, jnp.float32)
    it = jnp.full((BM, 1, 1), -1, jnp.int32)
    cols = []
    for _ in range(K):
        act = (pv > t) | ((pv == t) & (pi > it))
        m = jnp.min(jnp.where(act, pv, inf), axis=(1, 2), keepdims=True)
        ij = jnp.min(jnp.where(act & (pv == m), pi, N), axis=(1, 2),
                     keepdims=True)
        cols.append(ij.reshape(BM, 1))
        t, it = m, ij
    cand = jnp.concatenate(cols, axis=1)
    v = d2_ref[...]
    rest_less = (v < t) | ((v == t) & (idx3 < it))
    pool_less = (pv < t) | ((pv == t) & (pi < it))
    cnt = (jnp.sum(rest_less.astype(jnp.int32), axis=(1, 2), keepdims=True)
           + jnp.sum(pool_less.astype(jnp.int32), axis=(1, 2), keepdims=True))
    all_ok = jnp.all(cnt == K - 1)

    def _fallback():
        v = d2_ref[...]
        for r in range(RP):
            v = jnp.where(iota_s == pss[r], pvs[r], v)
        cols = []
        for _ in range(K):
            m = jnp.min(v, axis=(1, 2), keepdims=True)
            ij = jnp.min(jnp.where(v == m, idx3, N), axis=(1, 2),
                         keepdims=True)
            cols.append(ij.reshape(BM, 1))
            v = jnp.where(idx3 == ij, inf, v)
        return jnp.concatenate(cols, axis=1)

    idx_ref[...] = lax.cond(all_ok, lambda: cand, _fallback)


def _knn(p1p, p0tp):
    return pl.pallas_call(
        _knn_body,
        grid=(N // BM,),
        in_specs=[
            pl.BlockSpec((BM, 8), lambda i: (i, 0)),
            pl.BlockSpec((8, N), lambda i: (0, 0)),
        ],
        out_specs=pl.BlockSpec((BM, K), lambda i: (i, 0)),
        out_shape=jax.ShapeDtypeStruct((N, K), jnp.int32),
        scratch_shapes=[pltpu.VMEM((BM, NS, NL), jnp.float32)],
    )(p1p, p0tp)


TOT = N * K
CH = 512


def _gather_sc(table, idx_flat):
    info = plsc.get_sparse_core_info()
    nw = info.num_cores * info.num_subcores
    bpw = TOT // nw
    nch = bpw // CH
    mesh = plsc.VectorSubcoreMesh(core_axis_name="c", subcore_axis_name="s")

    @functools.partial(
        pl.kernel,
        mesh=mesh,
        out_type=jax.ShapeDtypeStruct((TOT, TW), jnp.float32),
        scratch_types=[
            pltpu.VMEM((CH,), jnp.int32),
            pltpu.VMEM((CH, TW), jnp.float32),
            pltpu.SemaphoreType.DMA,
        ],
    )
    def k(table_hbm, idx_hbm, out_hbm, idx_v, rows_v, sem):
        wid = lax.axis_index("s") * info.num_cores + lax.axis_index("c")
        base0 = wid * bpw

        def body(i, carry):
            base = base0 + i * CH
            pltpu.sync_copy(idx_hbm.at[pl.ds(base, CH)], idx_v)
            pltpu.async_copy(table_hbm.at[idx_v], rows_v, sem).wait()
            pltpu.sync_copy(rows_v, out_hbm.at[pl.ds(base, CH)])
            return carry

        lax.fori_loop(0, nch, body, 0)

    return k(table, idx_flat)


BM3 = 512


def _mlp_body(g_ref, p1_ref, feat_ref, wbig_ref, w3p_ref, wf_ref,
              wh1f_ref, wh1h_ref, c_ref, out_ref):
    gb = g_ref[...]
    y = jnp.dot(gb, wbig_ref[...], preferred_element_type=jnp.float32)
    p1b = p1_ref[...]
    pcon = jnp.dot(p1b, w3p_ref[...], preferred_element_type=jnp.float32)
    coords = gb[:, 64:67].reshape(BM3, K, 3)
    rela = coords - p1b[:, :3][:, None, :]
    dist = jnp.sqrt(jnp.sum(rela * rela, axis=-1, keepdims=True))
    wdist = c_ref[1:2, :].reshape(1, 1, 192)
    y3 = y.reshape(BM3, K, 192) - pcon[:, None, :] + dist * wdist
    ymax = jnp.max(y3, axis=1) + c_ref[0:1, :]
    featb = feat_ref[...]
    frz = jnp.dot(featb, wf_ref[...], preferred_element_type=jnp.float32)
    gate_r = jax.nn.sigmoid(ymax[:, 0:64] + frz[:, 0:64])
    gate_z = jax.nn.sigmoid(ymax[:, 64:128] + frz[:, 64:128])
    h10 = ymax[:, 128:192]
    h11 = jnp.tanh(
        jnp.dot(featb, wh1f_ref[...], preferred_element_type=jnp.float32)
        + jnp.dot(gate_r * h10, wh1h_ref[...], preferred_element_type=jnp.float32)
        + c_ref[2:3, 0:64])
    out_ref[...] = gate_z * h10 + (1.0 - gate_z) * h11


def _mlp(g, p1p, feat_t, wbig, w3p, wf, wh1f, wh1h, consts):
    return pl.pallas_call(
        _mlp_body,
        grid=(N // BM3,),
        in_specs=[
            pl.BlockSpec((BM3 * K, TW), lambda i: (i, 0)),
            pl.BlockSpec((BM3, 8), lambda i: (i, 0)),
            pl.BlockSpec((BM3, FEAT), lambda i: (i, 0)),
            pl.BlockSpec((TW, 192), lambda i: (0, 0)),
            pl.BlockSpec((8, 192), lambda i: (0, 0)),
            pl.BlockSpec((FEAT, 128), lambda i: (0, 0)),
            pl.BlockSpec((64, 64), lambda i: (0, 0)),
            pl.BlockSpec((64, 64), lambda i: (0, 0)),
            pl.BlockSpec((8, 192), lambda i: (0, 0)),
        ],
        out_specs=pl.BlockSpec((BM3, HID), lambda i: (i, 0)),
        out_shape=jax.ShapeDtypeStruct((N, HID), jnp.float32),
    )(g, p1p, feat_t, wbig, w3p, wf, wh1f, wh1h, consts)


def _pack_weights(W_R, b_R, W_Z, b_Z, W_H0, b_H0, W_H1, b_H1):
    wbig = jnp.zeros((TW, 192), jnp.float32)
    wbig = wbig.at[0:64, 0:64].set(W_R[4:68])
    wbig = wbig.at[64:67, 0:64].set(W_R[0:3])
    wbig = wbig.at[0:64, 64:128].set(W_Z[4:68])
    wbig = wbig.at[64:67, 64:128].set(W_Z[0:3])
    wbig = wbig.at[0:64, 128:192].set(W_H0[4:68])
    wbig = wbig.at[64:67, 128:192].set(W_H0[0:3])
    w3p = jnp.zeros((8, 192), jnp.float32)
    w3p = w3p.at[0:3, 0:64].set(W_R[0:3])
    w3p = w3p.at[0:3, 64:128].set(W_Z[0:3])
    w3p = w3p.at[0:3, 128:192].set(W_H0[0:3])
    wdist = jnp.concatenate([W_R[3], W_Z[3], W_H0[3]], axis=0)
    bcat = jnp.concatenate([b_R, b_Z, b_H0], axis=0)
    consts = jnp.zeros((8, 192), jnp.float32)
    consts = consts.at[0, :].set(bcat)
    consts = consts.at[1, :].set(wdist)
    consts = consts.at[2, 0:64].set(b_H1)
    wf = jnp.concatenate([W_R[68:132], W_Z[68:132]], axis=1)
    wh1f = W_H1[0:64]
    wh1h = W_H1[64:128]
    return wbig, w3p, wf, wh1f, wh1h, consts


def kernel(H0, points0, points1, contents1, motions1,
           W_R, b_R, W_Z, b_Z, W_H0, b_H0, W_H1, b_H1):
    p0t = points0[0]
    p1 = points1[0].T
    p1p = jnp.concatenate([p1, jnp.zeros((N, 5), jnp.float32)], axis=1)
    p0tp = jnp.concatenate([p0t, jnp.zeros((5, N), jnp.float32)], axis=0)

    nn_idx = _knn(p1p, p0tp)

    table = jnp.concatenate(
        [H0[0].T, p0t.T, jnp.zeros((N, TW - HID - 3), jnp.float32)], axis=1)
    g = _gather_sc(table, nn_idx.reshape(-1))

    feat_t = jnp.concatenate([contents1[0], motions1[0]], axis=0).T
    wbig, w3p, wf, wh1f, wh1h, consts = _pack_weights(
        W_R, b_R, W_Z, b_Z, W_H0, b_H0, W_H1, b_H1)
    out = _mlp(g, p1p, feat_t, wbig, w3p, wf, wh1f, wh1h, consts)
    return out.T[None]


# speedup vs baseline: 2.7550x; 1.9162x over previous
import functools

import jax
import jax.numpy as jnp
from jax import lax
from jax.experimental import pallas as pl
from jax.experimental.pallas import tpu as pltpu
from jax.experimental.pallas import tpu_sc as plsc

N = 8192
K = 16
HID = 64
FEAT = 64
TW = 128


BM = 128


NT = 64
NL = 128
RP = 4


def _knn_body(p1_ref, p0t_ref, idx_ref, d2_ref):
    p1 = p1_ref[...]
    p0t = p0t_ref[...]
    dot = jnp.dot(p1, p0t, preferred_element_type=jnp.float32)
    n1 = jnp.sum(p1 * p1, axis=1, keepdims=True)
    n0 = jnp.sum(p0t * p0t, axis=0, keepdims=True)
    d2_ref[...] = n1 + n0 - 2.0 * dot
    inf = jnp.float32(jnp.inf)
    iota_l = lax.broadcasted_iota(jnp.int32, (BM, NL), 1)
    pool_v, pool_i, winners = [], [], []
    for _ in range(RP):
        vs = [d2_ref[:, t * NL:(t + 1) * NL] for t in range(NT)]
        is_ = [jnp.full((BM, NL), t, jnp.int32) for t in range(NT)]
        while len(vs) > 1:
            nvs, nis = [], []
            for a in range(0, len(vs), 2):
                c = vs[a] <= vs[a + 1]
                nvs.append(jnp.where(c, vs[a], vs[a + 1]))
                nis.append(jnp.where(c, is_[a], is_[a + 1]))
            vs, is_ = nvs, nis
        lmv, lmi = vs[0], is_[0]
        pool_v.append(lmv)
        pool_i.append(lmi * NL + iota_l)
        winners.append(lmi)
        for t in range(NT):
            d2_ref[:, t * NL:(t + 1) * NL] = jnp.where(
                lmi == t, inf, d2_ref[:, t * NL:(t + 1) * NL])
    pv = jnp.concatenate(pool_v, axis=1)
    pi = jnp.concatenate(pool_i, axis=1)
    t = jnp.full((BM, 1), -inf, jnp.float32)
    it = jnp.full((BM, 1), -1, jnp.int32)
    cols = []
    for _ in range(K):
        act = (pv > t) | ((pv == t) & (pi > it))
        m = jnp.min(jnp.where(act, pv, inf), axis=1, keepdims=True)
        ij = jnp.min(jnp.where(act & (pv == m), pi, N), axis=1,
                     keepdims=True)
        cols.append(ij)
        t, it = m, ij
    cand = jnp.concatenate(cols, axis=1)
    v = d2_ref[...]
    less = (jnp.sum((v < t).astype(jnp.int32), axis=1, keepdims=True)
            + jnp.sum((pv < t).astype(jnp.int32), axis=1, keepdims=True))
    eq = (jnp.sum((v == t).astype(jnp.int32), axis=1, keepdims=True)
          + jnp.sum((pv == t).astype(jnp.int32), axis=1, keepdims=True))
    all_ok = jnp.all((less == K - 1) & (eq == 1))

    def _fallback():
        for r in range(RP):
            for tt in range(NT):
                d2_ref[:, tt * NL:(tt + 1) * NL] = jnp.where(
                    winners[r] == tt, pool_v[r],
                    d2_ref[:, tt * NL:(tt + 1) * NL])
        iota = lax.broadcasted_iota(jnp.int32, (BM, N), 1)
        v = d2_ref[...]
        cols = []
        for _ in range(K):
            m = jnp.min(v, axis=1, keepdims=True)
            ij = jnp.min(jnp.where(v == m, iota, N), axis=1, keepdims=True)
            cols.append(ij)
            v = jnp.where(iota == ij, inf, v)
        return jnp.concatenate(cols, axis=1)

    idx_ref[...] = lax.cond(all_ok, lambda: cand, _fallback)


def _knn(p1p, p0tp):
    return pl.pallas_call(
        _knn_body,
        grid=(N // BM,),
        in_specs=[
            pl.BlockSpec((BM, 8), lambda i: (i, 0)),
            pl.BlockSpec((8, N), lambda i: (0, 0)),
        ],
        out_specs=pl.BlockSpec((BM, K), lambda i: (i, 0)),
        out_shape=jax.ShapeDtypeStruct((N, K), jnp.int32),
        scratch_shapes=[pltpu.VMEM((BM, N), jnp.float32)],
    )(p1p, p0tp)


TOT = N * K
CH = 512


def _gather_sc(table, idx_flat):
    info = plsc.get_sparse_core_info()
    nw = info.num_cores * info.num_subcores
    bpw = TOT // nw
    nch = bpw // CH
    mesh = plsc.VectorSubcoreMesh(core_axis_name="c", subcore_axis_name="s")

    @functools.partial(
        pl.kernel,
        mesh=mesh,
        out_type=jax.ShapeDtypeStruct((TOT, TW), jnp.float32),
        scratch_types=[
            pltpu.VMEM((CH,), jnp.int32),
            pltpu.VMEM((CH, TW), jnp.float32),
            pltpu.SemaphoreType.DMA,
        ],
    )
    def k(table_hbm, idx_hbm, out_hbm, idx_v, rows_v, sem):
        wid = lax.axis_index("s") * info.num_cores + lax.axis_index("c")
        base0 = wid * bpw

        def body(i, carry):
            base = base0 + i * CH
            pltpu.sync_copy(idx_hbm.at[pl.ds(base, CH)], idx_v)
            pltpu.async_copy(table_hbm.at[idx_v], rows_v, sem).wait()
            pltpu.sync_copy(rows_v, out_hbm.at[pl.ds(base, CH)])
            return carry

        lax.fori_loop(0, nch, body, 0)

    return k(table, idx_flat)


BM3 = 512


def _mlp_body(g_ref, p1_ref, feat_ref, wbig_ref, w3p_ref, wf_ref,
              wh1f_ref, wh1h_ref, c_ref, out_ref):
    gb = g_ref[...]
    y = jnp.dot(gb, wbig_ref[...], preferred_element_type=jnp.float32)
    p1b = p1_ref[...]
    pcon = jnp.dot(p1b, w3p_ref[...], preferred_element_type=jnp.float32)
    coords = gb[:, 64:67].reshape(BM3, K, 3)
    rela = coords - p1b[:, :3][:, None, :]
    dist = jnp.sqrt(jnp.sum(rela * rela, axis=-1, keepdims=True))
    wdist = c_ref[1:2, :].reshape(1, 1, 192)
    y3 = y.reshape(BM3, K, 192) - pcon[:, None, :] + dist * wdist
    ymax = jnp.max(y3, axis=1) + c_ref[0:1, :]
    featb = feat_ref[...]
    frz = jnp.dot(featb, wf_ref[...], preferred_element_type=jnp.float32)
    gate_r = jax.nn.sigmoid(ymax[:, 0:64] + frz[:, 0:64])
    gate_z = jax.nn.sigmoid(ymax[:, 64:128] + frz[:, 64:128])
    h10 = ymax[:, 128:192]
    h11 = jnp.tanh(
        jnp.dot(featb, wh1f_ref[...], preferred_element_type=jnp.float32)
        + jnp.dot(gate_r * h10, wh1h_ref[...], preferred_element_type=jnp.float32)
        + c_ref[2:3, 0:64])
    out_ref[...] = gate_z * h10 + (1.0 - gate_z) * h11


def _mlp(g, p1p, feat_t, wbig, w3p, wf, wh1f, wh1h, consts):
    return pl.pallas_call(
        _mlp_body,
        grid=(N // BM3,),
        in_specs=[
            pl.BlockSpec((BM3 * K, TW), lambda i: (i, 0)),
            pl.BlockSpec((BM3, 8), lambda i: (i, 0)),
            pl.BlockSpec((BM3, FEAT), lambda i: (i, 0)),
            pl.BlockSpec((TW, 192), lambda i: (0, 0)),
            pl.BlockSpec((8, 192), lambda i: (0, 0)),
            pl.BlockSpec((FEAT, 128), lambda i: (0, 0)),
            pl.BlockSpec((64, 64), lambda i: (0, 0)),
            pl.BlockSpec((64, 64), lambda i: (0, 0)),
            pl.BlockSpec((8, 192), lambda i: (0, 0)),
        ],
        out_specs=pl.BlockSpec((BM3, HID), lambda i: (i, 0)),
        out_shape=jax.ShapeDtypeStruct((N, HID), jnp.float32),
    )(g, p1p, feat_t, wbig, w3p, wf, wh1f, wh1h, consts)


def _pack_weights(W_R, b_R, W_Z, b_Z, W_H0, b_H0, W_H1, b_H1):
    wbig = jnp.zeros((TW, 192), jnp.float32)
    wbig = wbig.at[0:64, 0:64].set(W_R[4:68])
    wbig = wbig.at[64:67, 0:64].set(W_R[0:3])
    wbig = wbig.at[0:64, 64:128].set(W_Z[4:68])
    wbig = wbig.at[64:67, 64:128].set(W_Z[0:3])
    wbig = wbig.at[0:64, 128:192].set(W_H0[4:68])
    wbig = wbig.at[64:67, 128:192].set(W_H0[0:3])
    w3p = jnp.zeros((8, 192), jnp.float32)
    w3p = w3p.at[0:3, 0:64].set(W_R[0:3])
    w3p = w3p.at[0:3, 64:128].set(W_Z[0:3])
    w3p = w3p.at[0:3, 128:192].set(W_H0[0:3])
    wdist = jnp.concatenate([W_R[3], W_Z[3], W_H0[3]], axis=0)
    bcat = jnp.concatenate([b_R, b_Z, b_H0], axis=0)
    consts = jnp.zeros((8, 192), jnp.float32)
    consts = consts.at[0, :].set(bcat)
    consts = consts.at[1, :].set(wdist)
    consts = consts.at[2, 0:64].set(b_H1)
    wf = jnp.concatenate([W_R[68:132], W_Z[68:132]], axis=1)
    wh1f = W_H1[0:64]
    wh1h = W_H1[64:128]
    return wbig, w3p, wf, wh1f, wh1h, consts


def kernel(H0, points0, points1, contents1, motions1,
           W_R, b_R, W_Z, b_Z, W_H0, b_H0, W_H1, b_H1):
    p0t = points0[0]
    p1 = points1[0].T
    p1p = jnp.concatenate([p1, jnp.zeros((N, 5), jnp.float32)], axis=1)
    p0tp = jnp.concatenate([p0t, jnp.zeros((5, N), jnp.float32)], axis=0)

    nn_idx = _knn(p1p, p0tp)

    table = jnp.concatenate(
        [H0[0].T, p0t.T, jnp.zeros((N, TW - HID - 3), jnp.float32)], axis=1)
    g = _gather_sc(table, nn_idx.reshape(-1))

    feat_t = jnp.concatenate([contents1[0], motions1[0]], axis=0).T
    wbig, w3p, wf, wh1f, wh1h, consts = _pack_weights(
        W_R, b_R, W_Z, b_Z, W_H0, b_H0, W_H1, b_H1)
    out = _mlp(g, p1p, feat_t, wbig, w3p, wf, wh1f, wh1h, consts)
    return out.T[None]


# speedup vs baseline: 2.7720x; 1.0062x over previous
import functools

import jax
import jax.numpy as jnp
from jax import lax
from jax.experimental import pallas as pl
from jax.experimental.pallas import tpu as pltpu
from jax.experimental.pallas import tpu_sc as plsc

N = 8192
K = 16
HID = 64
FEAT = 64
TW = 128


BM = 128


NT = 64
NL = 128
RP = 4


def _knn_body(p1_ref, p0t_ref, idx_ref, d2_ref):
    p1 = p1_ref[...]
    p0t = p0t_ref[...]
    dot = jnp.dot(p1, p0t, preferred_element_type=jnp.float32)
    n1 = jnp.sum(p1 * p1, axis=1, keepdims=True)
    n0 = jnp.sum(p0t * p0t, axis=0, keepdims=True)
    d2_ref[...] = n1 + n0 - 2.0 * dot
    inf = jnp.float32(jnp.inf)
    iota_l = lax.broadcasted_iota(jnp.int32, (BM, NL), 1)
    pool_v, pool_i, winners = [], [], []
    for _ in range(RP):
        vs = [d2_ref[:, t * NL:(t + 1) * NL] for t in range(NT)]
        is_ = [jnp.full((BM, NL), t, jnp.int32) for t in range(NT)]
        while len(vs) > 1:
            nvs, nis = [], []
            for a in range(0, len(vs), 2):
                c = vs[a] <= vs[a + 1]
                nvs.append(jnp.where(c, vs[a], vs[a + 1]))
                nis.append(jnp.where(c, is_[a], is_[a + 1]))
            vs, is_ = nvs, nis
        lmv, lmi = vs[0], is_[0]
        pool_v.append(lmv)
        pool_i.append(lmi * NL + iota_l)
        winners.append(lmi)
        for t in range(NT):
            d2_ref[:, t * NL:(t + 1) * NL] = jnp.where(
                lmi == t, inf, d2_ref[:, t * NL:(t + 1) * NL])
    pv = jnp.concatenate(pool_v, axis=1)
    pi = jnp.concatenate(pool_i, axis=1)
    t = jnp.full((BM, 1), -inf, jnp.float32)
    it = jnp.full((BM, 1), -1, jnp.int32)
    cols = []
    for _ in range(K):
        act = (pv > t) | ((pv == t) & (pi > it))
        m = jnp.min(jnp.where(act, pv, inf), axis=1, keepdims=True)
        ij = jnp.min(jnp.where(act & (pv == m), pi, N), axis=1,
                     keepdims=True)
        cols.append(ij)
        t, it = m, ij
    cand = jnp.concatenate(cols, axis=1)
    v = d2_ref[...]
    less = (jnp.sum((v < t).astype(jnp.int32), axis=1, keepdims=True)
            + jnp.sum((pv < t).astype(jnp.int32), axis=1, keepdims=True))
    eq = (jnp.sum((v == t).astype(jnp.int32), axis=1, keepdims=True)
          + jnp.sum((pv == t).astype(jnp.int32), axis=1, keepdims=True))
    all_ok = jnp.all((less == K - 1) & (eq == 1))

    def _fallback():
        for r in range(RP):
            for tt in range(NT):
                d2_ref[:, tt * NL:(tt + 1) * NL] = jnp.where(
                    winners[r] == tt, pool_v[r],
                    d2_ref[:, tt * NL:(tt + 1) * NL])
        iota = lax.broadcasted_iota(jnp.int32, (BM, N), 1)
        v = d2_ref[...]
        cols = []
        for _ in range(K):
            m = jnp.min(v, axis=1, keepdims=True)
            ij = jnp.min(jnp.where(v == m, iota, N), axis=1, keepdims=True)
            cols.append(ij)
            v = jnp.where(iota == ij, inf, v)
        return jnp.concatenate(cols, axis=1)

    idx_ref[...] = lax.cond(all_ok, lambda: cand, _fallback)


def _knn(p1p, p0tp):
    return pl.pallas_call(
        _knn_body,
        grid=(N // BM,),
        in_specs=[
            pl.BlockSpec((BM, 8), lambda i: (i, 0)),
            pl.BlockSpec((8, N), lambda i: (0, 0)),
        ],
        out_specs=pl.BlockSpec((BM, K), lambda i: (i, 0)),
        out_shape=jax.ShapeDtypeStruct((N, K), jnp.int32),
        scratch_shapes=[pltpu.VMEM((BM, N), jnp.float32)],
    )(p1p, p0tp)


TOT = N * K
CH = 256


def _gather_sc(table, idx_flat):
    info = plsc.get_sparse_core_info()
    nw = info.num_cores * info.num_subcores
    bpw = TOT // nw
    nch = bpw // CH
    mesh = plsc.VectorSubcoreMesh(core_axis_name="c", subcore_axis_name="s")

    @functools.partial(
        pl.kernel,
        mesh=mesh,
        out_type=jax.ShapeDtypeStruct((TOT, TW), jnp.float32),
        scratch_types=[
            pltpu.VMEM((bpw,), jnp.int32),
            pltpu.VMEM((2, CH, TW), jnp.float32),
            pltpu.SemaphoreType.DMA,
            pltpu.SemaphoreType.DMA,
            pltpu.SemaphoreType.DMA,
            pltpu.SemaphoreType.DMA,
        ],
    )
    def k(table_hbm, idx_hbm, out_hbm, idx_v, rows_v, sg0, sg1, so0, so1):
        wid = lax.axis_index("s") * info.num_cores + lax.axis_index("c")
        base0 = wid * bpw
        pltpu.sync_copy(idx_hbm.at[pl.ds(base0, bpw)], idx_v)
        sg = [sg0, sg1]
        so = [so0, so1]
        gather = [None, None]
        store = [None, None]
        for c in range(nch):
            b = c & 1
            if store[b] is not None:
                store[b].wait()
            gather[b] = pltpu.async_copy(
                table_hbm.at[idx_v.at[pl.ds(c * CH, CH)]], rows_v.at[b],
                sg[b])
            if c >= 1:
                pb = (c - 1) & 1
                gather[pb].wait()
                store[pb] = pltpu.async_copy(
                    rows_v.at[pb],
                    out_hbm.at[pl.ds(base0 + (c - 1) * CH, CH)], so[pb])
        lb = (nch - 1) & 1
        gather[lb].wait()
        store[lb] = pltpu.async_copy(
            rows_v.at[lb], out_hbm.at[pl.ds(base0 + (nch - 1) * CH, CH)],
            so[lb])
        store[(nch - 2) & 1].wait()
        store[lb].wait()

    return k(table, idx_flat)


BM3 = 512


def _mlp_body(g_ref, p1_ref, feat_ref, wbig_ref, w3p_ref, wf_ref,
              wh1f_ref, wh1h_ref, c_ref, out_ref):
    gb = g_ref[...]
    y = jnp.dot(gb, wbig_ref[...], preferred_element_type=jnp.float32)
    p1b = p1_ref[...]
    pcon = jnp.dot(p1b, w3p_ref[...], preferred_element_type=jnp.float32)
    coords = gb[:, 64:67].reshape(BM3, K, 3)
    rela = coords - p1b[:, :3][:, None, :]
    dist = jnp.sqrt(jnp.sum(rela * rela, axis=-1, keepdims=True))
    wdist = c_ref[1:2, :].reshape(1, 1, 192)
    y3 = y.reshape(BM3, K, 192) - pcon[:, None, :] + dist * wdist
    ymax = jnp.max(y3, axis=1) + c_ref[0:1, :]
    featb = feat_ref[...]
    frz = jnp.dot(featb, wf_ref[...], preferred_element_type=jnp.float32)
    gate_r = jax.nn.sigmoid(ymax[:, 0:64] + frz[:, 0:64])
    gate_z = jax.nn.sigmoid(ymax[:, 64:128] + frz[:, 64:128])
    h10 = ymax[:, 128:192]
    h11 = jnp.tanh(
        jnp.dot(featb, wh1f_ref[...], preferred_element_type=jnp.float32)
        + jnp.dot(gate_r * h10, wh1h_ref[...], preferred_element_type=jnp.float32)
        + c_ref[2:3, 0:64])
    out_ref[...] = gate_z * h10 + (1.0 - gate_z) * h11


def _mlp(g, p1p, feat_t, wbig, w3p, wf, wh1f, wh1h, consts):
    return pl.pallas_call(
        _mlp_body,
        grid=(N // BM3,),
        in_specs=[
            pl.BlockSpec((BM3 * K, TW), lambda i: (i, 0)),
            pl.BlockSpec((BM3, 8), lambda i: (i, 0)),
            pl.BlockSpec((BM3, FEAT), lambda i: (i, 0)),
            pl.BlockSpec((TW, 192), lambda i: (0, 0)),
            pl.BlockSpec((8, 192), lambda i: (0, 0)),
            pl.BlockSpec((FEAT, 128), lambda i: (0, 0)),
            pl.BlockSpec((64, 64), lambda i: (0, 0)),
            pl.BlockSpec((64, 64), lambda i: (0, 0)),
            pl.BlockSpec((8, 192), lambda i: (0, 0)),
        ],
        out_specs=pl.BlockSpec((BM3, HID), lambda i: (i, 0)),
        out_shape=jax.ShapeDtypeStruct((N, HID), jnp.float32),
    )(g, p1p, feat_t, wbig, w3p, wf, wh1f, wh1h, consts)


def _pack_weights(W_R, b_R, W_Z, b_Z, W_H0, b_H0, W_H1, b_H1):
    wbig = jnp.zeros((TW, 192), jnp.float32)
    wbig = wbig.at[0:64, 0:64].set(W_R[4:68])
    wbig = wbig.at[64:67, 0:64].set(W_R[0:3])
    wbig = wbig.at[0:64, 64:128].set(W_Z[4:68])
    wbig = wbig.at[64:67, 64:128].set(W_Z[0:3])
    wbig = wbig.at[0:64, 128:192].set(W_H0[4:68])
    wbig = wbig.at[64:67, 128:192].set(W_H0[0:3])
    w3p = jnp.zeros((8, 192), jnp.float32)
    w3p = w3p.at[0:3, 0:64].set(W_R[0:3])
    w3p = w3p.at[0:3, 64:128].set(W_Z[0:3])
    w3p = w3p.at[0:3, 128:192].set(W_H0[0:3])
    wdist = jnp.concatenate([W_R[3], W_Z[3], W_H0[3]], axis=0)
    bcat = jnp.concatenate([b_R, b_Z, b_H0], axis=0)
    consts = jnp.zeros((8, 192), jnp.float32)
    consts = consts.at[0, :].set(bcat)
    consts = consts.at[1, :].set(wdist)
    consts = consts.at[2, 0:64].set(b_H1)
    wf = jnp.concatenate([W_R[68:132], W_Z[68:132]], axis=1)
    wh1f = W_H1[0:64]
    wh1h = W_H1[64:128]
    return wbig, w3p, wf, wh1f, wh1h, consts


def kernel(H0, points0, points1, contents1, motions1,
           W_R, b_R, W_Z, b_Z, W_H0, b_H0, W_H1, b_H1):
    p0t = points0[0]
    p1 = points1[0].T
    p1p = jnp.concatenate([p1, jnp.zeros((N, 5), jnp.float32)], axis=1)
    p0tp = jnp.concatenate([p0t, jnp.zeros((5, N), jnp.float32)], axis=0)

    nn_idx = _knn(p1p, p0tp)

    table = jnp.concatenate(
        [H0[0].T, p0t.T, jnp.zeros((N, TW - HID - 3), jnp.float32)], axis=1)
    g = _gather_sc(table, nn_idx.reshape(-1))

    feat_t = jnp.concatenate([contents1[0], motions1[0]], axis=0).T
    wbig, w3p, wf, wh1f, wh1h, consts = _pack_weights(
        W_R, b_R, W_Z, b_Z, W_H0, b_H0, W_H1, b_H1)
    out = _mlp(g, p1p, feat_t, wbig, w3p, wf, wh1f, wh1h, consts)
    return out.T[None]
